# baseline (speedup 1.0000x reference)
import jax
import jax.numpy as jnp
from jax.experimental import pallas as pl


def _sage(x, edge_index, Wl, bl, Wr):
    src = edge_index[0]
    dst = edge_index[1]
    n = x.shape[0]
    msg = x[src]
    agg = jax.ops.segment_sum(msg, dst, num_segments=n)
    deg = jax.ops.segment_sum(jnp.ones((edge_index.shape[1],), x.dtype), dst, num_segments=n)
    mean = agg / jnp.clip(deg, 1.0)[:, None]
    return mean @ Wl + bl + x @ Wr


def _bn(x, gamma, beta):
    mu = jnp.mean(x, axis=0)
    var = jnp.mean((x - mu) ** 2, axis=0)
    return gamma * (x - mu) / jnp.sqrt(var + 1e-5) + beta


def _classifier_body(h_ref, wc_ref, bc_ref, out_ref):
    logits = h_ref[...] @ wc_ref[...] + bc_ref[...]
    m = jnp.max(logits, axis=1, keepdims=True)
    z = logits - m
    lse = jnp.log(jnp.sum(jnp.exp(z), axis=1, keepdims=True))
    out_ref[...] = z - lse


def kernel(x, edge_index, Wl0, bl0, Wr0, gamma0, beta0, Wl1, bl1, Wr1, gamma1, beta1, Wl2, bl2, Wr2, gamma2, beta2, Wc, bc):
    params = [(Wl0, bl0, Wr0, gamma0, beta0), (Wl1, bl1, Wr1, gamma1, beta1), (Wl2, bl2, Wr2, gamma2, beta2)]
    h = x
    for (Wl, bl, Wr, g, b) in params:
        h = _sage(h, edge_index, Wl, bl, Wr)
        h = _bn(h, g, b)
        h = jax.nn.relu(h)
    n = h.shape[0]
    c = Wc.shape[1]
    return pl.pallas_call(
        _classifier_body,
        out_shape=jax.ShapeDtypeStruct((n, c), jnp.float32),
    )(h, Wc, bc.reshape(1, c))


# speedup vs baseline: 3.9638x; 3.9638x over previous
import functools

import jax
import jax.numpy as jnp
from jax import lax
from jax.experimental import pallas as pl
from jax.experimental.pallas import tpu as pltpu
from jax.experimental.pallas import tpu_sc as plsc

NC, NS, LANES = 2, 16, 16
NW = NC * NS
CHUNK = 128


def _zero_rows(ref, nrows, width):
    def body(i, _):
        for j in range(width // LANES):
            ref[i, pl.ds(j * LANES, LANES)] = jnp.zeros((LANES,), jnp.float32)
        return 0

    lax.fori_loop(0, nrows, body, 0, unroll=False)


def _segsum_body(compute_deg, feat, rows_per_w, agg_rows,
                 h, srcm, dstm, *rest):
    if compute_deg:
        out_agg, out_deg, agg_sp, deg_sp, rows_v, srcv, dstv, ones_v, degz, sem = rest
    else:
        out_agg, agg_sp, rows_v, srcv, dstv, sem = rest

    c = lax.axis_index("c")
    s = lax.axis_index("s")
    wid = s * NC + c

    zrows = agg_rows // NS
    _zero_rows(rows_v, CHUNK, feat)
    base_z = s * zrows
    off = 0
    while off < zrows:
        sz = min(CHUNK, zrows - off)
        pltpu.sync_copy(rows_v.at[pl.ds(0, sz)], agg_sp.at[pl.ds(base_z + off, sz)])
        off += sz
    if compute_deg:
        _zero_rows(degz, zrows, LANES)
        pltpu.sync_copy(degz, deg_sp.at[pl.ds(base_z, zrows)])

        def fill_ones(i, _):
            ones_v[i, :] = jnp.ones((LANES,), jnp.float32)
            return 0

        lax.fori_loop(0, CHUNK, fill_ones, 0, unroll=False)
    plsc.subcore_barrier()

    def step(i, _):
        row = wid * rows_per_w + i
        pltpu.sync_copy(srcm.at[row], srcv)
        pltpu.sync_copy(dstm.at[row], dstv)
        pltpu.async_copy(h.at[srcv], rows_v, sem).wait()
        pltpu.sync_copy(rows_v, agg_sp.at[dstv], add=True)
        if compute_deg:
            pltpu.sync_copy(ones_v, deg_sp.at[dstv], add=True)
        return 0

    lax.fori_loop(0, rows_per_w, step, 0, unroll=False)
    plsc.subcore_barrier()

    orows = agg_rows // NS
    base_o = s * orows
    off = 0
    while off < orows:
        sz = min(CHUNK, orows - off)
        pltpu.sync_copy(agg_sp.at[pl.ds(base_o + off, sz)], rows_v.at[pl.ds(0, sz)])
        pltpu.sync_copy(rows_v.at[pl.ds(0, sz)], out_agg.at[c, pl.ds(base_o + off, sz)])
        off += sz
    if compute_deg:
        pltpu.sync_copy(deg_sp.at[pl.ds(base_o, orows)], degz.at[pl.ds(0, orows)])
        pltpu.sync_copy(degz.at[pl.ds(0, orows)], out_deg.at[c, pl.ds(base_o, orows)])


def _segsum(h, srcm, dstm, compute_deg):
    n_nodes, feat = h.shape
    erows = srcm.shape[0]
    rows_per_w = erows // NW
    agg_rows = -(-(n_nodes + 1) // (NS * 8)) * (NS * 8)
    assert erows % NW == 0

    mesh = plsc.VectorSubcoreMesh(
        core_axis_name="c", subcore_axis_name="s", num_cores=NC, num_subcores=NS
    )
    out_type = [jax.ShapeDtypeStruct((NC, agg_rows, feat), jnp.float32)]
    scratch = [
        pltpu.VMEM_SHARED((agg_rows, feat), jnp.float32),
        pltpu.VMEM((CHUNK, feat), jnp.float32),
        pltpu.VMEM((CHUNK,), jnp.int32),
        pltpu.VMEM((CHUNK,), jnp.int32),
        pltpu.SemaphoreType.DMA,
    ]
    if compute_deg:
        out_type.append(jax.ShapeDtypeStruct((NC, agg_rows, LANES), jnp.float32))
        scratch = (
            scratch[:1]
            + [pltpu.VMEM_SHARED((agg_rows, LANES), jnp.float32)]
            + scratch[1:4]
            + [
                pltpu.VMEM((CHUNK, LANES), jnp.float32),
                pltpu.VMEM((agg_rows // NS, LANES), jnp.float32),
            ]
            + scratch[4:]
        )

    body = functools.partial(_segsum_body, compute_deg, feat,
                             rows_per_w, agg_rows)
    fn = pl.kernel(body, out_type=tuple(out_type), mesh=mesh,
                   scratch_types=tuple(scratch),
                   compiler_params=pltpu.CompilerParams(use_tc_tiling_on_sc=False))
    return fn(h, srcm, dstm)


def _sage_block(h, aggp, degp, wl, bl, wr, g, b):
    n = h.shape[0]
    dp = degp[...]
    deg = dp[0, :n, 0:1] + dp[1, :n, 0:1]
    inv = 1.0 / jnp.maximum(deg, 1.0)
    mean = (aggp[0, :n] + aggp[1, :n]) * inv
    lin = (
        jnp.dot(mean, wl[...], preferred_element_type=jnp.float32)
        + bl[...]
        + jnp.dot(h[...], wr[...], preferred_element_type=jnp.float32)
    )
    mu = jnp.mean(lin, axis=0, keepdims=True)
    xc = lin - mu
    var = jnp.mean(xc * xc, axis=0, keepdims=True)
    y = g[...] * xc * lax.rsqrt(var + 1e-5) + b[...]
    return jnp.maximum(y, 0.0)


def _layer_mid_body(h, aggp, degp, wl, bl, wr, g, b, out):
    out[...] = _sage_block(h, aggp, degp, wl, bl, wr, g, b)


def _layer_final_body(h, aggp, degp, wl, bl, wr, g, b, wc, bc, out):
    hr = _sage_block(h, aggp, degp, wl, bl, wr, g, b)
    logits = jnp.dot(hr, wc[...], preferred_element_type=jnp.float32) + bc[...]
    m = jnp.max(logits, axis=1, keepdims=True)
    z = logits - m
    lse = jnp.log(jnp.sum(jnp.exp(z), axis=1, keepdims=True))
    out[...] = z - lse


def _layer_mid(h, aggp, degp, wl, bl, wr, g, b):
    n, feat = h.shape
    return pl.pallas_call(
        _layer_mid_body,
        out_shape=jax.ShapeDtypeStruct((n, feat), jnp.float32),
    )(h, aggp, degp, wl, bl.reshape(1, -1), wr, g.reshape(1, -1), b.reshape(1, -1))


def _layer_final(h, aggp, degp, wl, bl, wr, g, b, wc, bc):
    n = h.shape[0]
    ncls = wc.shape[1]
    return pl.pallas_call(
        _layer_final_body,
        out_shape=jax.ShapeDtypeStruct((n, ncls), jnp.float32),
    )(h, aggp, degp, wl, bl.reshape(1, -1), wr, g.reshape(1, -1), b.reshape(1, -1),
      wc, bc.reshape(1, -1))


def kernel(x, edge_index, Wl0, bl0, Wr0, gamma0, beta0, Wl1, bl1, Wr1, gamma1, beta1, Wl2, bl2, Wr2, gamma2, beta2, Wc, bc):
    n_nodes = x.shape[0]
    e = edge_index.shape[1]
    epad = -(-e // (CHUNK * NW)) * (CHUNK * NW)
    src = edge_index[0]
    dst = edge_index[1]
    if epad > e:
        pad = epad - e
        src = jnp.concatenate([src, jnp.zeros((pad,), jnp.int32)])
        dst = jnp.concatenate([dst, jnp.full((pad,), n_nodes, jnp.int32)])
    srcm = src.reshape(epad // CHUNK, CHUNK)
    dstm = dst.reshape(epad // CHUNK, CHUNK)

    agg0, degp = _segsum(x, srcm, dstm, compute_deg=True)
    h1 = _layer_mid(x, agg0, degp, Wl0, bl0, Wr0, gamma0, beta0)
    (agg1,) = _segsum(h1, srcm, dstm, compute_deg=False)
    h2 = _layer_mid(h1, agg1, degp, Wl1, bl1, Wr1, gamma1, beta1)
    (agg2,) = _segsum(h2, srcm, dstm, compute_deg=False)
    return _layer_final(h2, agg2, degp, Wl2, bl2, Wr2, gamma2, beta2, Wc, bc)
